# Initial kernel scaffold; baseline (speedup 1.0000x reference)
import jax
import jax.numpy as jnp
from jax.experimental import pallas as pl


def kernel(X, index, pe):
    raise NotImplementedError("write your pallas kernel here")


# speedup vs baseline: 3.0651x; 3.0651x over previous
import jax
import jax.numpy as jnp
from jax import lax
from jax.experimental import pallas as pl
from jax.experimental.pallas import tpu as pltpu
from jax.experimental.pallas import tpu_sc as plsc

_D = 64
_BN = 16384
_L = 50
_T = _BN * _L
_NC = 2
_NS = 16
_NW = _NC * _NS
_TPW = _T // _NW
_C = 512
_G = 128
_NCHUNK = _TPW // _C


def _sc_body(x_hbm, idx_hbm, pe_hbm, out_hbm, xbuf, gbuf, ibuf, xsem, gsem):
    wid = lax.axis_index("s") * _NC + lax.axis_index("c")
    base = wid * _TPW

    def chunk(ci, carry):
        t0 = base + ci * _C
        pltpu.sync_copy(idx_hbm.at[pl.ds(t0, _C)], ibuf)
        xcp = pltpu.async_copy(x_hbm.at[pl.ds(t0, _C)], xbuf, xsem)
        gcps = [
            pltpu.async_copy(
                pe_hbm.at[ibuf.at[pl.ds(j * _G, _G)]],
                gbuf.at[pl.ds(j * _G, _G)],
                gsem,
            )
            for j in range(_C // _G)
        ]
        xcp.wait()
        for cp in gcps:
            cp.wait()

        def row(r, c2):
            for d in range(_D // 16):
                plsc.addupdate(xbuf.at[r, pl.ds(16 * d, 16)],
                               gbuf[r, pl.ds(16 * d, 16)])
            return c2

        lax.fori_loop(0, _C, row, 0)
        pltpu.sync_copy(xbuf, out_hbm.at[pl.ds(t0, _C)])
        return carry

    lax.fori_loop(0, _NCHUNK, chunk, 0)


def kernel(X, index, pe):
    x2 = X.reshape(_T, _D)
    idx = index.reshape(_T)
    mesh = plsc.VectorSubcoreMesh(core_axis_name="c", subcore_axis_name="s")
    out = pl.kernel(
        _sc_body,
        out_type=jax.ShapeDtypeStruct((_T, _D), jnp.float32),
        mesh=mesh,
        compiler_params=pltpu.CompilerParams(use_tc_tiling_on_sc=False),
        scratch_types=[
            pltpu.VMEM((_C, _D), jnp.float32),
            pltpu.VMEM((_C, _D), jnp.float32),
            pltpu.VMEM((_C,), jnp.int32),
            pltpu.SemaphoreType.DMA,
            pltpu.SemaphoreType.DMA,
        ],
    )(x2, idx, pe)
    return out.reshape(_BN, _L, _D)
